# Initial kernel scaffold; baseline (speedup 1.0000x reference)
import jax
import jax.numpy as jnp
from jax.experimental import pallas as pl


def kernel(hidden_states, weight, bias):
    raise NotImplementedError("write your pallas kernel here")


# speedup vs baseline: 6.5024x; 6.5024x over previous
import functools

import jax
import jax.numpy as jnp
from jax import lax
from jax.experimental import pallas as pl
from jax.experimental.pallas import tpu as pltpu

_TOP_K = 8
_E = 64
_H = 2048
_N = 8192
_BLK = 1024


def _router_body(hs_ref, w_ref, b_ref, scores_ref, idx_ref, cnt_ref):
    i = pl.program_id(0)
    hs = hs_ref[...]
    w = w_ref[...]
    logits = lax.dot_general(
        hs, w, (((1,), (1,)), ((), ())), preferred_element_type=jnp.float32
    )
    logits = logits + b_ref[...]

    lane = lax.broadcasted_iota(jnp.int32, (_BLK, _E), 1)
    kcol = lax.broadcasted_iota(jnp.int32, (_BLK, _TOP_K), 1)
    avail = logits
    selected = jnp.zeros((_BLK, _E), jnp.bool_)
    idx_acc = jnp.zeros((_BLK, _TOP_K), jnp.int32)
    for k in range(_TOP_K):
        m = jnp.max(avail, axis=1, keepdims=True)
        sel_idx = jnp.min(jnp.where(avail == m, lane, _E), axis=1, keepdims=True)
        onehot = lane == sel_idx
        selected = selected | onehot
        idx_acc = idx_acc + jnp.where(kcol == k, sel_idx, 0)
        avail = jnp.where(onehot, -jnp.inf, avail)

    row_max = jnp.max(logits, axis=1, keepdims=True)
    num = jnp.where(selected, jnp.exp(logits - row_max), 0.0)
    denom = jnp.sum(num, axis=1, keepdims=True)
    scores_ref[...] = num / denom
    idx_ref[...] = idx_acc

    @pl.when(i == 0)
    def _init():
        cnt_ref[...] = jnp.zeros_like(cnt_ref)

    cnt_ref[...] += jnp.sum(selected.astype(jnp.int32), axis=0, keepdims=True)


@jax.jit
def kernel(hidden_states, weight, bias):
    hs = hidden_states.reshape(-1, _H)
    n = hs.shape[0]
    grid = (n // _BLK,)
    scores, idx, cnt = pl.pallas_call(
        _router_body,
        grid=grid,
        in_specs=[
            pl.BlockSpec((_BLK, _H), lambda i: (i, 0)),
            pl.BlockSpec((_E, _H), lambda i: (0, 0)),
            pl.BlockSpec((1, _E), lambda i: (0, 0)),
        ],
        out_specs=[
            pl.BlockSpec((_BLK, _E), lambda i: (i, 0)),
            pl.BlockSpec((_BLK, _TOP_K), lambda i: (i, 0)),
            pl.BlockSpec((1, _E), lambda i: (0, 0)),
        ],
        out_shape=[
            jax.ShapeDtypeStruct((n, _E), jnp.float32),
            jax.ShapeDtypeStruct((n, _TOP_K), jnp.int32),
            jax.ShapeDtypeStruct((1, _E), jnp.int32),
        ],
        compiler_params=pltpu.CompilerParams(
            dimension_semantics=("arbitrary",),
        ),
    )(hs, weight, bias.reshape(1, _E))
    return scores, idx, cnt.reshape(_E)
